# baseline (speedup 1.0000x reference)
import functools

import jax
import jax.numpy as jnp
from jax import lax
from jax.experimental import pallas as pl
from jax.experimental.pallas import tpu as pltpu
from jax.experimental.pallas import tpu_sc as plsc

_VOCAB = 100000
_D = 64
_DP = 128
_BATCH = 4096
_SEQ = 200

_NW = 32
_SPW = _BATCH // _NW
_IPW = _SPW * _SEQ
_G0 = 128
_G1 = _SEQ - _G0
_NBUF = 2


def _build(interpret=False):
  mesh = plsc.VectorSubcoreMesh(core_axis_name="c", subcore_axis_name="s")
  nc = 2

  @functools.partial(
      pl.kernel,
      out_type=jax.ShapeDtypeStruct((_BATCH, _SEQ, _D), jnp.float32),
      mesh=mesh,
      scratch_types=[
          pltpu.VMEM((_IPW,), jnp.int32),
          pltpu.VMEM_SHARED((_SEQ, _DP), jnp.float32),
          pltpu.VMEM((_NBUF, _SEQ, _DP), jnp.float32),
      ] + [pltpu.SemaphoreType.DMA] * (2 * _NBUF),
      compiler_params=pltpu.CompilerParams(use_tc_tiling_on_sc=False),
      interpret=interpret,
  )
  def k(table_hbm, idx_hbm, pos_hbm, out_hbm, idx_v, pos_v, bufs, *sems):
    gsems = sems[:_NBUF]
    wsems = sems[_NBUF:]
    sid = lax.axis_index("s")
    wid = sid * nc + lax.axis_index("c")
    base = wid * _SPW

    pltpu.sync_copy(idx_hbm.at[pl.ds(wid * _IPW, _IPW)], idx_v)
    @pl.when(sid == 0)
    def _():
      pltpu.sync_copy(pos_hbm, pos_v)
    plsc.subcore_barrier()

    def stage(s, b):
      pltpu.sync_copy(pos_v, bufs.at[b])
      pltpu.async_copy(
          table_hbm.at[idx_v.at[pl.ds(s * _SEQ, _G0)]],
          bufs.at[b, pl.ds(0, _G0)], gsems[b], add=True)
      pltpu.async_copy(
          table_hbm.at[idx_v.at[pl.ds(s * _SEQ + _G0, _G1)]],
          bufs.at[b, pl.ds(_G0, _G1)], gsems[b], add=True)

    for b in range(_NBUF):
      stage(b, b)

    def body(i, carry):
      s0 = i * _NBUF
      for b in range(_NBUF):
        s = s0 + b
        pltpu.make_async_copy(
            table_hbm.at[pl.ds(0, _SEQ)], bufs.at[b], gsems[b]).wait()
        pltpu.async_copy(
            bufs.at[b, :, pl.ds(0, _D)], out_hbm.at[base + s], wsems[b])

        @pl.when(s + _NBUF < _SPW)
        def _():
          pltpu.make_async_copy(
              bufs.at[b, :, pl.ds(0, _D)], out_hbm.at[0], wsems[b]).wait()
          stage(s + _NBUF, b)

      return carry

    lax.fori_loop(0, _SPW // _NBUF, body, 0)

    for b in range(_NBUF):
      pltpu.make_async_copy(
          bufs.at[b, :, pl.ds(0, _D)], out_hbm.at[0], wsems[b]).wait()

  return k


_kernel_call = _build()


def kernel(x, word_emb, pos_emb):
  wpad = jnp.pad(word_emb, ((0, 0), (0, _DP - _D)))
  ppad = jnp.pad(pos_emb[:_SEQ], ((0, 0), (0, _DP - _D)))
  xf = x.astype(jnp.int32).reshape(_BATCH * _SEQ)
  return _kernel_call(wpad, xf, ppad)


# speedup vs baseline: 1.1635x; 1.1635x over previous
import functools

import jax
import jax.numpy as jnp
from jax import lax
from jax.experimental import pallas as pl
from jax.experimental.pallas import tpu as pltpu
from jax.experimental.pallas import tpu_sc as plsc

_VOCAB = 100000
_D = 64
_BATCH = 4096
_SEQ = 200

_NW = 32
_SPW = _BATCH // _NW
_IPW = _SPW * _SEQ
_G0 = 128
_G1 = _SEQ - _G0
_NBUF = 4


def _build(interpret=False):
  mesh = plsc.VectorSubcoreMesh(core_axis_name="c", subcore_axis_name="s")
  nc = 2

  @functools.partial(
      pl.kernel,
      out_type=jax.ShapeDtypeStruct((_BATCH, _SEQ, _D), jnp.float32),
      mesh=mesh,
      scratch_types=[
          pltpu.VMEM((_IPW,), jnp.int32),
          pltpu.VMEM_SHARED((_SEQ, _D), jnp.float32),
          pltpu.VMEM((_NBUF, _SEQ, _D), jnp.float32),
      ] + [pltpu.SemaphoreType.DMA] * (2 * _NBUF),
      compiler_params=pltpu.CompilerParams(use_tc_tiling_on_sc=False),
      interpret=interpret,
  )
  def k(table_hbm, idx_hbm, pos_hbm, out_hbm, idx_v, pos_v, bufs, *sems):
    gsems = sems[:_NBUF]
    wsems = sems[_NBUF:]
    sid = lax.axis_index("s")
    wid = sid * nc + lax.axis_index("c")
    base = wid * _SPW

    pltpu.sync_copy(idx_hbm.at[pl.ds(wid * _IPW, _IPW)], idx_v)
    @pl.when(sid == 0)
    def _():
      pltpu.sync_copy(pos_hbm, pos_v)
    plsc.subcore_barrier()

    def stage(s, b):
      pltpu.sync_copy(pos_v, bufs.at[b])
      pltpu.async_copy(
          table_hbm.at[idx_v.at[pl.ds(s * _SEQ, _G0)]],
          bufs.at[b, pl.ds(0, _G0)], gsems[b], add=True)
      pltpu.async_copy(
          table_hbm.at[idx_v.at[pl.ds(s * _SEQ + _G0, _G1)]],
          bufs.at[b, pl.ds(_G0, _G1)], gsems[b], add=True)

    for b in range(_NBUF):
      stage(b, b)

    def body(i, carry):
      s0 = i * _NBUF
      for b in range(_NBUF):
        s = s0 + b
        pltpu.make_async_copy(
            out_hbm.at[0], bufs.at[b], gsems[b]).wait()
        pltpu.async_copy(bufs.at[b], out_hbm.at[base + s], wsems[b])

        @pl.when(s + _NBUF < _SPW)
        def _():
          pltpu.make_async_copy(
              bufs.at[b], out_hbm.at[0], wsems[b]).wait()
          stage(s + _NBUF, b)

      return carry

    lax.fori_loop(0, _SPW // _NBUF, body, 0)

    for b in range(_NBUF):
      pltpu.make_async_copy(
          bufs.at[b], out_hbm.at[0], wsems[b]).wait()

  return k


_kernel_call = _build()


def kernel(x, word_emb, pos_emb):
  xf = x.astype(jnp.int32).reshape(_BATCH * _SEQ)
  return _kernel_call(word_emb, xf, pos_emb[:_SEQ])
